# baseline (speedup 1.0000x reference)
import jax
import jax.numpy as jnp
import numpy as np
from jax import lax
from jax.experimental import pallas as pl
from jax.experimental.pallas import tpu as pltpu
from jax.experimental.pallas import tpu_sc as plsc

N, E, B = 4096, 16384, 128
DIN, DE, DG, D, DEH, DGH = 128, 16, 256, 64, 128, 512
STEPS, S2S_ITERS = 3, 3

NC, NS = 2, 16
NW = NC * NS
EPW = E // NW
ICH = 128
NCH = EPW // ICH
RPZ = N // NS

_F32 = jnp.float32
_BF16 = jnp.bfloat16

_sc_mesh = plsc.VectorSubcoreMesh(core_axis_name="c", subcore_axis_name="s")

_RMAT = np.repeat(np.eye(D, dtype=np.float32), D, axis=1)
_ZEROS = np.zeros((N, 2 * D), np.float32)


NCHH = NCH // 2
EPWH = EPW // 2


def _sc_gather_body(h_hbm, src_hbm, out_hbm, idx_v, rows_v, sem):
    wid = lax.axis_index("s") * NC + lax.axis_index("c")
    pltpu.sync_copy(src_hbm.at[pl.ds(wid * NCHH, NCHH)], idx_v)
    copies = [
        pltpu.async_copy(h_hbm.at[idx_v.at[j]], rows_v.at[pl.ds(j * ICH, ICH)], sem)
        for j in range(NCHH)
    ]
    for c in copies:
        c.wait()
    pltpu.sync_copy(rows_v, out_hbm.at[pl.ds(wid * EPWH, EPWH)])


def _gather(h, src2d_half):
    fn = pl.kernel(
        _sc_gather_body,
        out_type=jax.ShapeDtypeStruct((EH, 2 * D), _F32),
        mesh=_sc_mesh,
        scratch_types=[
            pltpu.VMEM((NCHH, ICH), jnp.int32),
            pltpu.VMEM((EPWH, 2 * D), _F32),
            pltpu.SemaphoreType.DMA,
        ],
    )
    return fn(h, src2d_half)


def _sc_scatter_body(msg_hbm, dst_hbm, zero_hbm, out_hbm, idx_v, rows_v, acc_sh, sem):
    cid = lax.axis_index("c")
    sid = lax.axis_index("s")
    wid = sid * NC + cid
    loads = [
        pltpu.async_copy(zero_hbm.at[pl.ds(sid * RPZ, RPZ)],
                         acc_sh.at[pl.ds(sid * RPZ, RPZ)], sem),
        pltpu.async_copy(dst_hbm.at[pl.ds(wid * NCHH, NCHH)], idx_v, sem),
        pltpu.async_copy(msg_hbm.at[pl.ds(wid * EPWH, EPWH)], rows_v, sem),
    ]
    for c in loads:
        c.wait()
    plsc.subcore_barrier()
    adds = [
        pltpu.async_copy(rows_v.at[pl.ds(j * ICH, ICH)], acc_sh.at[idx_v.at[j]],
                         sem, add=True)
        for j in range(NCHH)
    ]
    for c in adds:
        c.wait()
    plsc.subcore_barrier()
    pltpu.sync_copy(acc_sh.at[pl.ds(sid * RPZ, RPZ)],
                    out_hbm.at[pl.ds(cid * N + sid * RPZ, RPZ)])


def _scatter(msg_half, dst2d_half, zeros_nd):
    fn = pl.kernel(
        _sc_scatter_body,
        out_type=jax.ShapeDtypeStruct((2 * N, 2 * D), _F32),
        mesh=_sc_mesh,
        scratch_types=[
            pltpu.VMEM((NCHH, ICH), jnp.int32),
            pltpu.VMEM((EPWH, 2 * D), _F32),
            pltpu.VMEM_SHARED((N, 2 * D), _F32),
            pltpu.SemaphoreType.DMA,
        ],
    )
    return fn(msg_half, dst2d_half, zeros_nd)


def _h0_body(nf_ref, wp_ref, bp_ref, hf_ref, hb_ref):
    h0 = jnp.maximum(
        jnp.dot(nf_ref[...], wp_ref[...], preferred_element_type=_F32)
        + bp_ref[...], 0.0)
    hf_ref[...] = h0
    hb_ref[...] = jnp.concatenate([h0, jnp.zeros_like(h0)], axis=1)


def _h0(node_feats, wp, bp):
    return pl.pallas_call(
        _h0_body,
        out_shape=(jax.ShapeDtypeStruct((N, D), _F32),
                   jax.ShapeDtypeStruct((N, 2 * D), _F32)),
    )(node_feats, wp, bp)


ET = 512


def _msg1_body(ef_ref, we1_ref, be1_ref, w2_ref, b2_ref, hs_ref, rmat_ref,
               th_ref, out_ref):
    t = jnp.maximum(
        jnp.dot(ef_ref[...], we1_ref[...], preferred_element_type=_F32)
        + be1_ref[...], 0.0)
    th = jnp.dot(t, w2_ref[...], preferred_element_type=_F32) + b2_ref[...]
    th_ref[...] = th.astype(_BF16)
    hs = hs_ref[...][:, :D]
    hexp = jnp.dot(hs, rmat_ref[...], preferred_element_type=_F32)
    parts = [hexp[:, j * 2 * D:(j + 1) * 2 * D] * th[:, j * 2 * D:(j + 1) * 2 * D]
             for j in range(D // 2)]
    while len(parts) > 1:
        parts = [parts[i] + parts[i + 1] for i in range(0, len(parts), 2)]
    red = parts[0]
    msg = red[:, :D] + red[:, D:]
    out_ref[...] = jnp.concatenate([msg, jnp.zeros_like(msg)], axis=1)


def _msg1(edge_feats, we1, be1, w2, b2row, h_src_half, rmat, half):
    base = half * GT
    return pl.pallas_call(
        _msg1_body,
        grid=(GT,),
        in_specs=[pl.BlockSpec((ET, DE), lambda i, b=base: (i + b, 0)),
                  pl.BlockSpec((DE, DEH), lambda i: (0, 0)),
                  pl.BlockSpec((1, DEH), lambda i: (0, 0)),
                  pl.BlockSpec((DEH, D * D), lambda i: (0, 0)),
                  pl.BlockSpec((1, D * D), lambda i: (0, 0)),
                  pl.BlockSpec((ET, 2 * D), lambda i: (i, 0)),
                  pl.BlockSpec((D, D * D), lambda i: (0, 0))],
        out_specs=(pl.BlockSpec((ET, D * D), lambda i: (i, 0)),
                   pl.BlockSpec((ET, 2 * D), lambda i: (i, 0))),
        out_shape=(jax.ShapeDtypeStruct((EH, D * D), _BF16),
                   jax.ShapeDtypeStruct((EH, 2 * D), _F32)),
    )(edge_feats, we1, be1, w2, b2row, h_src_half, rmat)


EH = E // 2
GT = EH // ET


def _msg_body(hs_ref, th_ref, rmat_ref, out_ref):
    hs = hs_ref[...][:, :D]
    th = th_ref[...].astype(_F32)
    hexp = jnp.dot(hs, rmat_ref[...], preferred_element_type=_F32)
    parts = [hexp[:, j * 2 * D:(j + 1) * 2 * D] * th[:, j * 2 * D:(j + 1) * 2 * D]
             for j in range(D // 2)]
    while len(parts) > 1:
        parts = [parts[i] + parts[i + 1] for i in range(0, len(parts), 2)]
    red = parts[0]
    msg = red[:, :D] + red[:, D:]
    out_ref[...] = jnp.concatenate([msg, jnp.zeros_like(msg)], axis=1)


def _msg(h_src_half, theta_half, rmat):
    return pl.pallas_call(
        _msg_body,
        grid=(GT,),
        in_specs=[pl.BlockSpec((ET, 2 * D), lambda i: (i, 0)),
                  pl.BlockSpec((ET, D * D), lambda i: (i, 0)),
                  pl.BlockSpec((D, D * D), lambda i: (0, 0))],
        out_specs=pl.BlockSpec((ET, 2 * D), lambda i: (i, 0)),
        out_shape=jax.ShapeDtypeStruct((EH, 2 * D), _F32),
    )(h_src_half, theta_half, rmat)


RT = 1024


def _gru_body(a0_ref, a1_ref, a2_ref, a3_ref, h_ref, wir, wiz, win, whr, whz, whn,
              bc, br, bz, bi_n, bh_n, hf_ref, hb_ref):
    m = jnp.maximum(a0_ref[...][:, :D] + a1_ref[...][:, :D]
                    + a2_ref[...][:, :D] + a3_ref[...][:, :D] + bc[...], 0.0)
    hv = h_ref[...]
    r = jax.nn.sigmoid(jnp.dot(m, wir[...], preferred_element_type=_F32)
                       + jnp.dot(hv, whr[...], preferred_element_type=_F32) + br[...])
    z = jax.nn.sigmoid(jnp.dot(m, wiz[...], preferred_element_type=_F32)
                       + jnp.dot(hv, whz[...], preferred_element_type=_F32) + bz[...])
    hn = jnp.dot(hv, whn[...], preferred_element_type=_F32) + bh_n[...]
    n = jnp.tanh(jnp.dot(m, win[...], preferred_element_type=_F32) + bi_n[...] + r * hn)
    hnew = (1.0 - z) * n + z * hv
    hf_ref[...] = hnew
    hb_ref[...] = jnp.concatenate([hnew, jnp.zeros_like(hnew)], axis=1)


def _gru(agg2a, agg2b, h, gw):
    full = pl.BlockSpec((D, D), lambda i: (0, 0))
    bias = pl.BlockSpec((1, D), lambda i: (0, 0))
    row = pl.BlockSpec((RT, D), lambda i: (i, 0))
    row2 = pl.BlockSpec((RT, 2 * D), lambda i: (i, 0))
    nsteps = N // RT
    row2hi = pl.BlockSpec((RT, 2 * D), lambda i: (i + nsteps, 0))
    return pl.pallas_call(
        _gru_body,
        grid=(nsteps,),
        in_specs=[row2, row2hi, row2, row2hi, row, full, full, full, full, full, full,
                  bias, bias, bias, bias, bias],
        out_specs=(row, row2),
        out_shape=(jax.ShapeDtypeStruct((N, D), _F32),
                   jax.ShapeDtypeStruct((N, 2 * D), _F32)),
    )(agg2a, agg2a, agg2b, agg2b, h, gw['wir'], gw['wiz'], gw['win'], gw['whr'], gw['whz'],
      gw['whn'], gw['bc'], gw['br'], gw['bz'], gw['bi_n'], gw['bh_n'])


_SW_KEYS = (
    ['wi0_' + g for g in 'ifgo'] + ['wh0_' + g for g in 'ifgo'] + ['b0_' + g for g in 'ifgo']
    + ['wi1_' + g for g in 'ifgo'] + ['wh1_' + g for g in 'ifgo'] + ['b1_' + g for g in 'ifgo']
    + ['wg1', 'bg1', 'wg2', 'bg2', 'wp1a', 'wp1b', 'bp1', 'wp2', 'bp2'])


def _s2s_body(h_ref, ida_ref, idb_ref, gf_ref, *rest):
    w_ref = dict(zip(_SW_KEYS, rest[:-1]))
    out_ref = rest[-1]
    h = h_ref[...]
    ida = ida_ref[...]
    idb = idb_ref[...]
    lane_b = lax.broadcasted_iota(jnp.int32, (N, B), 1)
    mask = (lane_b == ida).astype(_F32)
    sub_b = lax.broadcasted_iota(jnp.int32, (B, N), 0)
    maskt = (sub_b == idb).astype(_F32)
    neg = jnp.float32(-1e30)

    def dot(a, b):
        return jnp.dot(a, b, preferred_element_type=_F32)

    hs0 = jnp.zeros((B, D), _F32)
    hs1 = jnp.zeros((B, D), _F32)
    cs0 = jnp.zeros((B, D), _F32)
    cs1 = jnp.zeros((B, D), _F32)
    q_star = jnp.zeros((B, 2 * D), _F32)

    for _ in range(S2S_ITERS):
        ig = dot(q_star, w_ref['wi0_i'][...]) + dot(hs0, w_ref['wh0_i'][...]) + w_ref['b0_i'][...]
        fg = dot(q_star, w_ref['wi0_f'][...]) + dot(hs0, w_ref['wh0_f'][...]) + w_ref['b0_f'][...]
        gg = dot(q_star, w_ref['wi0_g'][...]) + dot(hs0, w_ref['wh0_g'][...]) + w_ref['b0_g'][...]
        og = dot(q_star, w_ref['wi0_o'][...]) + dot(hs0, w_ref['wh0_o'][...]) + w_ref['b0_o'][...]
        cs0 = jax.nn.sigmoid(fg) * cs0 + jax.nn.sigmoid(ig) * jnp.tanh(gg)
        hs0 = jax.nn.sigmoid(og) * jnp.tanh(cs0)

        ig = dot(hs0, w_ref['wi1_i'][...]) + dot(hs1, w_ref['wh1_i'][...]) + w_ref['b1_i'][...]
        fg = dot(hs0, w_ref['wi1_f'][...]) + dot(hs1, w_ref['wh1_f'][...]) + w_ref['b1_f'][...]
        gg = dot(hs0, w_ref['wi1_g'][...]) + dot(hs1, w_ref['wh1_g'][...]) + w_ref['b1_g'][...]
        og = dot(hs0, w_ref['wi1_o'][...]) + dot(hs1, w_ref['wh1_o'][...]) + w_ref['b1_o'][...]
        cs1 = jax.nn.sigmoid(fg) * cs1 + jax.nn.sigmoid(ig) * jnp.tanh(gg)
        hs1 = jax.nn.sigmoid(og) * jnp.tanh(cs1)
        q = hs1

        q_g = dot(mask, q)
        e = jnp.sum(h * q_g, axis=1, keepdims=True)
        masked = jnp.where(mask > 0.0, e, neg)
        smax = jnp.max(masked, axis=0, keepdims=True)
        smax_g = jnp.sum(mask * smax, axis=1, keepdims=True)
        ex = jnp.exp(e - smax_g)
        den = dot(maskt, ex)
        den_g = dot(mask, den)
        alpha = ex / den_g
        readout = dot(maskt, alpha * h)
        q_star = jnp.concatenate([q, readout], axis=1)

    gfe = jnp.maximum(dot(gf_ref[...], w_ref['wg1'][...]) + w_ref['bg1'][...], 0.0)
    gfe = jnp.maximum(dot(gfe, w_ref['wg2'][...]) + w_ref['bg2'][...], 0.0)
    pre = jnp.maximum(dot(q_star, w_ref['wp1a'][...]) + dot(gfe, w_ref['wp1b'][...])
                      + w_ref['bp1'][...], 0.0)
    out_ref[...] = dot(pre, w_ref['wp2'][...]) + w_ref['bp2'][...]


def _s2s(h, ida, idb, g_feat, sw):
    return pl.pallas_call(
        _s2s_body,
        out_shape=jax.ShapeDtypeStruct((B, 1), _F32),
    )(h, ida, idb, g_feat, *[sw[k] for k in _SW_KEYS])


def kernel(node_feats, edge_feats, g_feat, params, edge_index, graph_ids):
    p = params
    src2d = edge_index[0].astype(jnp.int32).reshape(NW * NCH, ICH)
    dst2d = edge_index[1].astype(jnp.int32).reshape(NW * NCH, ICH)
    ida = graph_ids.astype(jnp.int32).reshape(N, 1)
    idb = graph_ids.astype(jnp.int32).reshape(1, N)

    wih_t = p['gru_Wih'].T
    whh_t = p['gru_Whh'].T
    gw = {
        'wir': wih_t[:, 0 * D:1 * D], 'wiz': wih_t[:, 1 * D:2 * D], 'win': wih_t[:, 2 * D:3 * D],
        'whr': whh_t[:, 0 * D:1 * D], 'whz': whh_t[:, 1 * D:2 * D], 'whn': whh_t[:, 2 * D:3 * D],
        'bc': p['b_conv'].reshape(1, D),
        'br': (p['gru_bih'][0 * D:1 * D] + p['gru_bhh'][0 * D:1 * D]).reshape(1, D),
        'bz': (p['gru_bih'][1 * D:2 * D] + p['gru_bhh'][1 * D:2 * D]).reshape(1, D),
        'bi_n': p['gru_bih'][2 * D:3 * D].reshape(1, D),
        'bh_n': p['gru_bhh'][2 * D:3 * D].reshape(1, D),
    }

    def lstm_split(wih, whh, bih, bhh, gate):
        g = {'i': 0, 'f': 1, 'g': 2, 'o': 3}[gate]
        return (wih[g * D:(g + 1) * D].T, whh[g * D:(g + 1) * D].T,
                (bih[g * D:(g + 1) * D] + bhh[g * D:(g + 1) * D]).reshape(1, D))

    sw = {}
    for gate in 'ifgo':
        wi, wh, b = lstm_split(p['lstm_Wih0'], p['lstm_Whh0'], p['lstm_bih0'], p['lstm_bhh0'], gate)
        sw['wi0_' + gate], sw['wh0_' + gate], sw['b0_' + gate] = wi, wh, b
        wi, wh, b = lstm_split(p['lstm_Wih1'], p['lstm_Whh1'], p['lstm_bih1'], p['lstm_bhh1'], gate)
        sw['wi1_' + gate], sw['wh1_' + gate], sw['b1_' + gate] = wi, wh, b
    sw['wg1'] = p['W_g1']
    sw['bg1'] = p['b_g1'].reshape(1, DGH)
    sw['wg2'] = p['W_g2']
    sw['bg2'] = p['b_g2'].reshape(1, DGH)
    sw['wp1a'] = p['W_p1'][:2 * D]
    sw['wp1b'] = p['W_p1'][2 * D:]
    sw['bp1'] = p['b_p1'].reshape(1, D)
    sw['wp2'] = p['W_p2']
    sw['bp2'] = p['b_p2'].reshape(1, 1)

    hf, hb = _h0(node_feats, p['W_proj'], p['b_proj'].reshape(1, D))

    rmat = _RMAT
    zeros_nd = _ZEROS
    srch = (src2d[:NW * NCHH], src2d[NW * NCHH:])
    dsth = (dst2d[:NW * NCHH], dst2d[NW * NCHH:])
    be1r = p['b_e1'].reshape(1, DEH)
    b2r = p['b_e2'].reshape(1, D * D)
    thetas = [None, None]
    for s in range(STEPS):
        hsA = _gather(hb, srch[0])
        hsB = _gather(hb, srch[1])
        if s == 0:
            thetas[0], msgA = _msg1(edge_feats, p['W_e1'], be1r, p['W_e2'], b2r,
                                    hsA, rmat, 0)
            thetas[1], msgB = _msg1(edge_feats, p['W_e1'], be1r, p['W_e2'], b2r,
                                    hsB, rmat, 1)
        else:
            msgA = _msg(hsA, thetas[0], rmat)
            msgB = _msg(hsB, thetas[1], rmat)
        aggA = _scatter(msgA, dsth[0], zeros_nd)
        aggB = _scatter(msgB, dsth[1], zeros_nd)
        hf, hb = _gru(aggA, aggB, hf, gw)

    return _s2s(hf, ida, idb, g_feat, sw)


# speedup vs baseline: 1.0224x; 1.0224x over previous
import jax
import jax.numpy as jnp
import numpy as np
from jax import lax
from jax.experimental import pallas as pl
from jax.experimental.pallas import tpu as pltpu
from jax.experimental.pallas import tpu_sc as plsc

N, E, B = 4096, 16384, 128
DIN, DE, DG, D, DEH, DGH = 128, 16, 256, 64, 128, 512
STEPS, S2S_ITERS = 3, 3

NC, NS = 2, 16
NW = NC * NS
EPW = E // NW
ICH = 128
NCH = EPW // ICH
RPZ = N // NS

_F32 = jnp.float32
_BF16 = jnp.bfloat16

_sc_mesh = plsc.VectorSubcoreMesh(core_axis_name="c", subcore_axis_name="s")

_RMAT = np.repeat(np.eye(D, dtype=np.float32), D, axis=1)
_ZEROS = np.zeros((N, 2 * D), np.float32)


def _sc_gather_body(h_hbm, src_hbm, out_hbm, idx_v, rows_v, sem):
    wid = lax.axis_index("s") * NC + lax.axis_index("c")
    pltpu.sync_copy(src_hbm.at[pl.ds(wid * NCH, NCH)], idx_v)
    copies = [
        pltpu.async_copy(h_hbm.at[idx_v.at[j]], rows_v.at[pl.ds(j * ICH, ICH)], sem)
        for j in range(NCH)
    ]
    for c in copies:
        c.wait()
    pltpu.sync_copy(rows_v, out_hbm.at[pl.ds(wid * EPW, EPW)])


def _gather(h, src2d):
    fn = pl.kernel(
        _sc_gather_body,
        out_type=jax.ShapeDtypeStruct((E, 2 * D), _F32),
        mesh=_sc_mesh,
        scratch_types=[
            pltpu.VMEM((NCH, ICH), jnp.int32),
            pltpu.VMEM((EPW, 2 * D), _F32),
            pltpu.SemaphoreType.DMA,
        ],
    )
    return fn(h, src2d)


def _sc_scatter_body(msg_hbm, dst_hbm, zero_hbm, out_hbm, idx_v, rows_v, acc_sh, sem):
    cid = lax.axis_index("c")
    sid = lax.axis_index("s")
    wid = sid * NC + cid
    loads = [
        pltpu.async_copy(zero_hbm.at[pl.ds(sid * RPZ, RPZ)],
                         acc_sh.at[pl.ds(sid * RPZ, RPZ)], sem),
        pltpu.async_copy(dst_hbm.at[pl.ds(wid * NCH, NCH)], idx_v, sem),
        pltpu.async_copy(msg_hbm.at[pl.ds(wid * EPW, EPW)], rows_v, sem),
    ]
    for c in loads:
        c.wait()
    plsc.subcore_barrier()
    adds = [
        pltpu.async_copy(rows_v.at[pl.ds(j * ICH, ICH)], acc_sh.at[idx_v.at[j]],
                         sem, add=True)
        for j in range(NCH)
    ]
    for c in adds:
        c.wait()
    plsc.subcore_barrier()
    pltpu.sync_copy(acc_sh.at[pl.ds(sid * RPZ, RPZ)],
                    out_hbm.at[pl.ds(cid * N + sid * RPZ, RPZ)])


def _scatter(msg, dst2d, zeros_nd):
    fn = pl.kernel(
        _sc_scatter_body,
        out_type=jax.ShapeDtypeStruct((2 * N, 2 * D), _F32),
        mesh=_sc_mesh,
        scratch_types=[
            pltpu.VMEM((NCH, ICH), jnp.int32),
            pltpu.VMEM((EPW, 2 * D), _F32),
            pltpu.VMEM_SHARED((N, 2 * D), _F32),
            pltpu.SemaphoreType.DMA,
        ],
    )
    return fn(msg, dst2d, zeros_nd)


def _h0_body(nf_ref, wp_ref, bp_ref, hf_ref, hb_ref):
    h0 = jnp.maximum(
        jnp.dot(nf_ref[...], wp_ref[...], preferred_element_type=_F32)
        + bp_ref[...], 0.0)
    hf_ref[...] = h0
    hb_ref[...] = jnp.concatenate([h0, jnp.zeros_like(h0)], axis=1)


def _h0(node_feats, wp, bp):
    return pl.pallas_call(
        _h0_body,
        out_shape=(jax.ShapeDtypeStruct((N, D), _F32),
                   jax.ShapeDtypeStruct((N, 2 * D), _F32)),
    )(node_feats, wp, bp)


ET = 512


def _msg1_body(ef_ref, we1_ref, be1_ref, w2_ref, b2_ref, hs_ref, rmat_ref,
               th_ref, out_ref):
    t = jnp.maximum(
        jnp.dot(ef_ref[...], we1_ref[...], preferred_element_type=_F32)
        + be1_ref[...], 0.0)
    th = jnp.dot(t, w2_ref[...], preferred_element_type=_F32) + b2_ref[...]
    th_ref[...] = th.astype(_BF16)
    hs = hs_ref[...][:, :D]
    hexp = jnp.dot(hs, rmat_ref[...], preferred_element_type=_F32)
    parts = [hexp[:, j * 2 * D:(j + 1) * 2 * D] * th[:, j * 2 * D:(j + 1) * 2 * D]
             for j in range(D // 2)]
    while len(parts) > 1:
        parts = [parts[i] + parts[i + 1] for i in range(0, len(parts), 2)]
    red = parts[0]
    msg = red[:, :D] + red[:, D:]
    out_ref[...] = jnp.concatenate([msg, jnp.zeros_like(msg)], axis=1)


def _msg1(edge_feats, we1, be1, w2, b2row, h_src, rmat):
    return pl.pallas_call(
        _msg1_body,
        grid=(E // ET,),
        in_specs=[pl.BlockSpec((ET, DE), lambda i: (i, 0)),
                  pl.BlockSpec((DE, DEH), lambda i: (0, 0)),
                  pl.BlockSpec((1, DEH), lambda i: (0, 0)),
                  pl.BlockSpec((DEH, D * D), lambda i: (0, 0)),
                  pl.BlockSpec((1, D * D), lambda i: (0, 0)),
                  pl.BlockSpec((ET, 2 * D), lambda i: (i, 0)),
                  pl.BlockSpec((D, D * D), lambda i: (0, 0))],
        out_specs=(pl.BlockSpec((ET, D * D), lambda i: (i, 0)),
                   pl.BlockSpec((ET, 2 * D), lambda i: (i, 0))),
        out_shape=(jax.ShapeDtypeStruct((E, D * D), _BF16),
                   jax.ShapeDtypeStruct((E, 2 * D), _F32)),
    )(edge_feats, we1, be1, w2, b2row, h_src, rmat)


def _msg_body(hs_ref, th_ref, rmat_ref, out_ref):
    hs = hs_ref[...][:, :D]
    th = th_ref[...].astype(_F32)
    hexp = jnp.dot(hs, rmat_ref[...], preferred_element_type=_F32)
    parts = [hexp[:, j * 2 * D:(j + 1) * 2 * D] * th[:, j * 2 * D:(j + 1) * 2 * D]
             for j in range(D // 2)]
    while len(parts) > 1:
        parts = [parts[i] + parts[i + 1] for i in range(0, len(parts), 2)]
    red = parts[0]
    msg = red[:, :D] + red[:, D:]
    out_ref[...] = jnp.concatenate([msg, jnp.zeros_like(msg)], axis=1)


def _msg(h_src, theta, rmat):
    return pl.pallas_call(
        _msg_body,
        grid=(E // ET,),
        in_specs=[pl.BlockSpec((ET, 2 * D), lambda i: (i, 0)),
                  pl.BlockSpec((ET, D * D), lambda i: (i, 0)),
                  pl.BlockSpec((D, D * D), lambda i: (0, 0))],
        out_specs=pl.BlockSpec((ET, 2 * D), lambda i: (i, 0)),
        out_shape=jax.ShapeDtypeStruct((E, 2 * D), _F32),
    )(h_src, theta, rmat)


RT = 1024


def _gru_body(a0_ref, a1_ref, h_ref, wir, wiz, win, whr, whz, whn,
              bc, br, bz, bi_n, bh_n, hf_ref, hb_ref):
    m = jnp.maximum(a0_ref[...][:, :D] + a1_ref[...][:, :D] + bc[...], 0.0)
    hv = h_ref[...]
    r = jax.nn.sigmoid(jnp.dot(m, wir[...], preferred_element_type=_F32)
                       + jnp.dot(hv, whr[...], preferred_element_type=_F32) + br[...])
    z = jax.nn.sigmoid(jnp.dot(m, wiz[...], preferred_element_type=_F32)
                       + jnp.dot(hv, whz[...], preferred_element_type=_F32) + bz[...])
    hn = jnp.dot(hv, whn[...], preferred_element_type=_F32) + bh_n[...]
    n = jnp.tanh(jnp.dot(m, win[...], preferred_element_type=_F32) + bi_n[...] + r * hn)
    hnew = (1.0 - z) * n + z * hv
    hf_ref[...] = hnew
    hb_ref[...] = jnp.concatenate([hnew, jnp.zeros_like(hnew)], axis=1)


def _gru(agg2, h, gw):
    full = pl.BlockSpec((D, D), lambda i: (0, 0))
    bias = pl.BlockSpec((1, D), lambda i: (0, 0))
    row = pl.BlockSpec((RT, D), lambda i: (i, 0))
    row2 = pl.BlockSpec((RT, 2 * D), lambda i: (i, 0))
    nsteps = N // RT
    row2hi = pl.BlockSpec((RT, 2 * D), lambda i: (i + nsteps, 0))
    return pl.pallas_call(
        _gru_body,
        grid=(nsteps,),
        in_specs=[row2, row2hi, row, full, full, full, full, full, full,
                  bias, bias, bias, bias, bias],
        out_specs=(row, row2),
        out_shape=(jax.ShapeDtypeStruct((N, D), _F32),
                   jax.ShapeDtypeStruct((N, 2 * D), _F32)),
    )(agg2, agg2, h, gw['wir'], gw['wiz'], gw['win'], gw['whr'], gw['whz'],
      gw['whn'], gw['bc'], gw['br'], gw['bz'], gw['bi_n'], gw['bh_n'])


_SW_KEYS = (
    ['wi0_' + g for g in 'ifgo'] + ['wh0_' + g for g in 'ifgo'] + ['b0_' + g for g in 'ifgo']
    + ['wi1_' + g for g in 'ifgo'] + ['wh1_' + g for g in 'ifgo'] + ['b1_' + g for g in 'ifgo']
    + ['wg1', 'bg1', 'wg2', 'bg2', 'wp1a', 'wp1b', 'bp1', 'wp2', 'bp2'])


def _s2s_body(h_ref, ida_ref, idb_ref, gf_ref, *rest):
    w_ref = dict(zip(_SW_KEYS, rest[:-1]))
    out_ref = rest[-1]
    h = h_ref[...]
    ida = ida_ref[...]
    idb = idb_ref[...]
    lane_b = lax.broadcasted_iota(jnp.int32, (N, B), 1)
    mask = (lane_b == ida).astype(_F32)
    sub_b = lax.broadcasted_iota(jnp.int32, (B, N), 0)
    maskt = (sub_b == idb).astype(_F32)
    neg = jnp.float32(-1e30)

    def dot(a, b):
        return jnp.dot(a, b, preferred_element_type=_F32)

    hs0 = jnp.zeros((B, D), _F32)
    hs1 = jnp.zeros((B, D), _F32)
    cs0 = jnp.zeros((B, D), _F32)
    cs1 = jnp.zeros((B, D), _F32)
    q_star = jnp.zeros((B, 2 * D), _F32)

    for _ in range(S2S_ITERS):
        ig = dot(q_star, w_ref['wi0_i'][...]) + dot(hs0, w_ref['wh0_i'][...]) + w_ref['b0_i'][...]
        fg = dot(q_star, w_ref['wi0_f'][...]) + dot(hs0, w_ref['wh0_f'][...]) + w_ref['b0_f'][...]
        gg = dot(q_star, w_ref['wi0_g'][...]) + dot(hs0, w_ref['wh0_g'][...]) + w_ref['b0_g'][...]
        og = dot(q_star, w_ref['wi0_o'][...]) + dot(hs0, w_ref['wh0_o'][...]) + w_ref['b0_o'][...]
        cs0 = jax.nn.sigmoid(fg) * cs0 + jax.nn.sigmoid(ig) * jnp.tanh(gg)
        hs0 = jax.nn.sigmoid(og) * jnp.tanh(cs0)

        ig = dot(hs0, w_ref['wi1_i'][...]) + dot(hs1, w_ref['wh1_i'][...]) + w_ref['b1_i'][...]
        fg = dot(hs0, w_ref['wi1_f'][...]) + dot(hs1, w_ref['wh1_f'][...]) + w_ref['b1_f'][...]
        gg = dot(hs0, w_ref['wi1_g'][...]) + dot(hs1, w_ref['wh1_g'][...]) + w_ref['b1_g'][...]
        og = dot(hs0, w_ref['wi1_o'][...]) + dot(hs1, w_ref['wh1_o'][...]) + w_ref['b1_o'][...]
        cs1 = jax.nn.sigmoid(fg) * cs1 + jax.nn.sigmoid(ig) * jnp.tanh(gg)
        hs1 = jax.nn.sigmoid(og) * jnp.tanh(cs1)
        q = hs1

        q_g = dot(mask, q)
        e = jnp.sum(h * q_g, axis=1, keepdims=True)
        masked = jnp.where(mask > 0.0, e, neg)
        smax = jnp.max(masked, axis=0, keepdims=True)
        smax_g = jnp.sum(mask * smax, axis=1, keepdims=True)
        ex = jnp.exp(e - smax_g)
        den = dot(maskt, ex)
        den_g = dot(mask, den)
        alpha = ex / den_g
        readout = dot(maskt, alpha * h)
        q_star = jnp.concatenate([q, readout], axis=1)

    gfe = jnp.maximum(dot(gf_ref[...], w_ref['wg1'][...]) + w_ref['bg1'][...], 0.0)
    gfe = jnp.maximum(dot(gfe, w_ref['wg2'][...]) + w_ref['bg2'][...], 0.0)
    pre = jnp.maximum(dot(q_star, w_ref['wp1a'][...]) + dot(gfe, w_ref['wp1b'][...])
                      + w_ref['bp1'][...], 0.0)
    out_ref[...] = dot(pre, w_ref['wp2'][...]) + w_ref['bp2'][...]


def _s2s(h, ida, idb, g_feat, sw):
    return pl.pallas_call(
        _s2s_body,
        out_shape=jax.ShapeDtypeStruct((B, 1), _F32),
    )(h, ida, idb, g_feat, *[sw[k] for k in _SW_KEYS])


def kernel(node_feats, edge_feats, g_feat, params, edge_index, graph_ids):
    p = params
    src2d = edge_index[0].astype(jnp.int32).reshape(NW * NCH, ICH)
    dst2d = edge_index[1].astype(jnp.int32).reshape(NW * NCH, ICH)
    ida = graph_ids.astype(jnp.int32).reshape(N, 1)
    idb = graph_ids.astype(jnp.int32).reshape(1, N)

    wih_t = p['gru_Wih'].T
    whh_t = p['gru_Whh'].T
    gw = {
        'wir': wih_t[:, 0 * D:1 * D], 'wiz': wih_t[:, 1 * D:2 * D], 'win': wih_t[:, 2 * D:3 * D],
        'whr': whh_t[:, 0 * D:1 * D], 'whz': whh_t[:, 1 * D:2 * D], 'whn': whh_t[:, 2 * D:3 * D],
        'bc': p['b_conv'].reshape(1, D),
        'br': (p['gru_bih'][0 * D:1 * D] + p['gru_bhh'][0 * D:1 * D]).reshape(1, D),
        'bz': (p['gru_bih'][1 * D:2 * D] + p['gru_bhh'][1 * D:2 * D]).reshape(1, D),
        'bi_n': p['gru_bih'][2 * D:3 * D].reshape(1, D),
        'bh_n': p['gru_bhh'][2 * D:3 * D].reshape(1, D),
    }

    def lstm_split(wih, whh, bih, bhh, gate):
        g = {'i': 0, 'f': 1, 'g': 2, 'o': 3}[gate]
        return (wih[g * D:(g + 1) * D].T, whh[g * D:(g + 1) * D].T,
                (bih[g * D:(g + 1) * D] + bhh[g * D:(g + 1) * D]).reshape(1, D))

    sw = {}
    for gate in 'ifgo':
        wi, wh, b = lstm_split(p['lstm_Wih0'], p['lstm_Whh0'], p['lstm_bih0'], p['lstm_bhh0'], gate)
        sw['wi0_' + gate], sw['wh0_' + gate], sw['b0_' + gate] = wi, wh, b
        wi, wh, b = lstm_split(p['lstm_Wih1'], p['lstm_Whh1'], p['lstm_bih1'], p['lstm_bhh1'], gate)
        sw['wi1_' + gate], sw['wh1_' + gate], sw['b1_' + gate] = wi, wh, b
    sw['wg1'] = p['W_g1']
    sw['bg1'] = p['b_g1'].reshape(1, DGH)
    sw['wg2'] = p['W_g2']
    sw['bg2'] = p['b_g2'].reshape(1, DGH)
    sw['wp1a'] = p['W_p1'][:2 * D]
    sw['wp1b'] = p['W_p1'][2 * D:]
    sw['bp1'] = p['b_p1'].reshape(1, D)
    sw['wp2'] = p['W_p2']
    sw['bp2'] = p['b_p2'].reshape(1, 1)

    hf, hb = _h0(node_feats, p['W_proj'], p['b_proj'].reshape(1, D))

    rmat = _RMAT
    zeros_nd = _ZEROS
    theta = None
    for s in range(STEPS):
        h_src = _gather(hb, src2d)
        if s == 0:
            theta, msg = _msg1(edge_feats, p['W_e1'], p['b_e1'].reshape(1, DEH),
                               p['W_e2'], p['b_e2'].reshape(1, D * D), h_src, rmat)
        else:
            msg = _msg(h_src, theta, rmat)
        agg2 = _scatter(msg, dst2d, zeros_nd)
        hf, hb = _gru(agg2, hf, gw)

    return _s2s(hf, ida, idb, g_feat, sw)


# speedup vs baseline: 1.1048x; 1.0806x over previous
import jax
import jax.numpy as jnp
import numpy as np
from jax import lax
from jax.experimental import pallas as pl
from jax.experimental.pallas import tpu as pltpu
from jax.experimental.pallas import tpu_sc as plsc

N, E, B = 4096, 16384, 128
DIN, DE, DG, D, DEH, DGH = 128, 16, 256, 64, 128, 512
STEPS, S2S_ITERS = 3, 3

NC, NS = 2, 16
NW = NC * NS
EPW = E // NW
ICH = 128
NCH = EPW // ICH
RPZ = N // NS

_F32 = jnp.float32
_BF16 = jnp.bfloat16

_sc_mesh = plsc.VectorSubcoreMesh(core_axis_name="c", subcore_axis_name="s")

_RMAT = np.repeat(np.eye(D, dtype=np.float32), D, axis=1)
_ZEROS = np.zeros((N, 2 * D), np.float32)


def _sc_gather_body(h_hbm, src_hbm, out_hbm, idx_v, rows_v, sem):
    wid = lax.axis_index("s") * NC + lax.axis_index("c")
    pltpu.sync_copy(src_hbm.at[pl.ds(wid * NCH, NCH)], idx_v)
    copies = [
        pltpu.async_copy(h_hbm.at[idx_v.at[j]], rows_v.at[pl.ds(j * ICH, ICH)], sem)
        for j in range(NCH)
    ]
    for c in copies:
        c.wait()
    pltpu.sync_copy(rows_v, out_hbm.at[pl.ds(wid * EPW, EPW)])


def _gather(h, src2d):
    fn = pl.kernel(
        _sc_gather_body,
        out_type=jax.ShapeDtypeStruct((E, 2 * D), _F32),
        mesh=_sc_mesh,
        scratch_types=[
            pltpu.VMEM((NCH, ICH), jnp.int32),
            pltpu.VMEM((EPW, 2 * D), _F32),
            pltpu.SemaphoreType.DMA,
        ],
    )
    return fn(h, src2d)


def _sc_scatter_body(msg_hbm, dst_hbm, zero_hbm, out_hbm, idx_v, rows_v, acc_sh, sem):
    cid = lax.axis_index("c")
    sid = lax.axis_index("s")
    wid = sid * NC + cid
    loads = [
        pltpu.async_copy(zero_hbm.at[pl.ds(sid * RPZ, RPZ)],
                         acc_sh.at[pl.ds(sid * RPZ, RPZ)], sem),
        pltpu.async_copy(dst_hbm.at[pl.ds(wid * NCH, NCH)], idx_v, sem),
        pltpu.async_copy(msg_hbm.at[pl.ds(wid * EPW, EPW)], rows_v, sem),
    ]
    for c in loads:
        c.wait()
    plsc.subcore_barrier()
    adds = [
        pltpu.async_copy(rows_v.at[pl.ds(j * ICH, ICH)], acc_sh.at[idx_v.at[j]],
                         sem, add=True)
        for j in range(NCH)
    ]
    for c in adds:
        c.wait()
    plsc.subcore_barrier()
    pltpu.sync_copy(acc_sh.at[pl.ds(sid * RPZ, RPZ)],
                    out_hbm.at[pl.ds(cid * N + sid * RPZ, RPZ)])


def _scatter(msg, dst2d, zeros_nd):
    fn = pl.kernel(
        _sc_scatter_body,
        out_type=jax.ShapeDtypeStruct((2 * N, 2 * D), _F32),
        mesh=_sc_mesh,
        scratch_types=[
            pltpu.VMEM((NCH, ICH), jnp.int32),
            pltpu.VMEM((EPW, 2 * D), _F32),
            pltpu.VMEM_SHARED((N, 2 * D), _F32),
            pltpu.SemaphoreType.DMA,
        ],
    )
    return fn(msg, dst2d, zeros_nd)


def _h0_body(nf_ref, wp_ref, bp_ref, hf_ref, hb_ref):
    h0 = jnp.maximum(
        jnp.dot(nf_ref[...], wp_ref[...], preferred_element_type=_F32)
        + bp_ref[...], 0.0)
    hf_ref[...] = h0
    hb_ref[...] = jnp.concatenate([h0, jnp.zeros_like(h0)], axis=1)


def _h0(node_feats, wp, bp):
    return pl.pallas_call(
        _h0_body,
        out_shape=(jax.ShapeDtypeStruct((N, D), _F32),
                   jax.ShapeDtypeStruct((N, 2 * D), _F32)),
    )(node_feats, wp, bp)


ET = 1024


def _msg1_body(ef_ref, we1_ref, be1_ref, w2_ref, b2_ref, hs_ref, rmat_ref,
               th_ref, out_ref):
    t = jnp.maximum(
        jnp.dot(ef_ref[...], we1_ref[...], preferred_element_type=_F32)
        + be1_ref[...], 0.0)
    th = jnp.dot(t, w2_ref[...], preferred_element_type=_F32) + b2_ref[...]
    th_ref[...] = th.astype(_BF16)
    hs = hs_ref[...][:, :D]
    hexp = jnp.dot(hs, rmat_ref[...], preferred_element_type=_F32)
    parts = [hexp[:, j * 2 * D:(j + 1) * 2 * D] * th[:, j * 2 * D:(j + 1) * 2 * D]
             for j in range(D // 2)]
    while len(parts) > 1:
        parts = [parts[i] + parts[i + 1] for i in range(0, len(parts), 2)]
    red = parts[0]
    msg = red[:, :D] + red[:, D:]
    out_ref[...] = jnp.concatenate([msg, jnp.zeros_like(msg)], axis=1)


def _msg1(edge_feats, we1, be1, w2, b2row, h_src, rmat):
    return pl.pallas_call(
        _msg1_body,
        grid=(E // ET,),
        in_specs=[pl.BlockSpec((ET, DE), lambda i: (i, 0)),
                  pl.BlockSpec((DE, DEH), lambda i: (0, 0)),
                  pl.BlockSpec((1, DEH), lambda i: (0, 0)),
                  pl.BlockSpec((DEH, D * D), lambda i: (0, 0)),
                  pl.BlockSpec((1, D * D), lambda i: (0, 0)),
                  pl.BlockSpec((ET, 2 * D), lambda i: (i, 0)),
                  pl.BlockSpec((D, D * D), lambda i: (0, 0))],
        out_specs=(pl.BlockSpec((ET, D * D), lambda i: (i, 0)),
                   pl.BlockSpec((ET, 2 * D), lambda i: (i, 0))),
        out_shape=(jax.ShapeDtypeStruct((E, D * D), _BF16),
                   jax.ShapeDtypeStruct((E, 2 * D), _F32)),
    )(edge_feats, we1, be1, w2, b2row, h_src, rmat)


def _msg_body(hs_ref, th_ref, rmat_ref, out_ref):
    hs = hs_ref[...][:, :D]
    th = th_ref[...].astype(_F32)
    hexp = jnp.dot(hs, rmat_ref[...], preferred_element_type=_F32)
    parts = [hexp[:, j * 2 * D:(j + 1) * 2 * D] * th[:, j * 2 * D:(j + 1) * 2 * D]
             for j in range(D // 2)]
    while len(parts) > 1:
        parts = [parts[i] + parts[i + 1] for i in range(0, len(parts), 2)]
    red = parts[0]
    msg = red[:, :D] + red[:, D:]
    out_ref[...] = jnp.concatenate([msg, jnp.zeros_like(msg)], axis=1)


def _msg(h_src, theta, rmat):
    return pl.pallas_call(
        _msg_body,
        grid=(E // ET,),
        in_specs=[pl.BlockSpec((ET, 2 * D), lambda i: (i, 0)),
                  pl.BlockSpec((ET, D * D), lambda i: (i, 0)),
                  pl.BlockSpec((D, D * D), lambda i: (0, 0))],
        out_specs=pl.BlockSpec((ET, 2 * D), lambda i: (i, 0)),
        out_shape=jax.ShapeDtypeStruct((E, 2 * D), _F32),
    )(h_src, theta, rmat)


RT = 1024


def _gru_body(a0_ref, a1_ref, h_ref, wir, wiz, win, whr, whz, whn,
              bc, br, bz, bi_n, bh_n, hf_ref, hb_ref):
    m = jnp.maximum(a0_ref[...][:, :D] + a1_ref[...][:, :D] + bc[...], 0.0)
    hv = h_ref[...]
    r = jax.nn.sigmoid(jnp.dot(m, wir[...], preferred_element_type=_F32)
                       + jnp.dot(hv, whr[...], preferred_element_type=_F32) + br[...])
    z = jax.nn.sigmoid(jnp.dot(m, wiz[...], preferred_element_type=_F32)
                       + jnp.dot(hv, whz[...], preferred_element_type=_F32) + bz[...])
    hn = jnp.dot(hv, whn[...], preferred_element_type=_F32) + bh_n[...]
    n = jnp.tanh(jnp.dot(m, win[...], preferred_element_type=_F32) + bi_n[...] + r * hn)
    hnew = (1.0 - z) * n + z * hv
    hf_ref[...] = hnew
    hb_ref[...] = jnp.concatenate([hnew, jnp.zeros_like(hnew)], axis=1)


def _gru(agg2, h, gw):
    full = pl.BlockSpec((D, D), lambda i: (0, 0))
    bias = pl.BlockSpec((1, D), lambda i: (0, 0))
    row = pl.BlockSpec((RT, D), lambda i: (i, 0))
    row2 = pl.BlockSpec((RT, 2 * D), lambda i: (i, 0))
    nsteps = N // RT
    row2hi = pl.BlockSpec((RT, 2 * D), lambda i: (i + nsteps, 0))
    return pl.pallas_call(
        _gru_body,
        grid=(nsteps,),
        in_specs=[row2, row2hi, row, full, full, full, full, full, full,
                  bias, bias, bias, bias, bias],
        out_specs=(row, row2),
        out_shape=(jax.ShapeDtypeStruct((N, D), _F32),
                   jax.ShapeDtypeStruct((N, 2 * D), _F32)),
    )(agg2, agg2, h, gw['wir'], gw['wiz'], gw['win'], gw['whr'], gw['whz'],
      gw['whn'], gw['bc'], gw['br'], gw['bz'], gw['bi_n'], gw['bh_n'])


_SW_KEYS = (
    ['wi0_' + g for g in 'ifgo'] + ['wh0_' + g for g in 'ifgo'] + ['b0_' + g for g in 'ifgo']
    + ['wi1_' + g for g in 'ifgo'] + ['wh1_' + g for g in 'ifgo'] + ['b1_' + g for g in 'ifgo']
    + ['wg1', 'bg1', 'wg2', 'bg2', 'wp1a', 'wp1b', 'bp1', 'wp2', 'bp2'])


def _s2s_body(h_ref, ida_ref, idb_ref, gf_ref, *rest):
    w_ref = dict(zip(_SW_KEYS, rest[:-1]))
    out_ref = rest[-1]
    h = h_ref[...]
    ida = ida_ref[...]
    idb = idb_ref[...]
    lane_b = lax.broadcasted_iota(jnp.int32, (N, B), 1)
    mask = (lane_b == ida).astype(_F32)
    sub_b = lax.broadcasted_iota(jnp.int32, (B, N), 0)
    maskt = (sub_b == idb).astype(_F32)
    neg = jnp.float32(-1e30)

    def dot(a, b):
        return jnp.dot(a, b, preferred_element_type=_F32)

    hs0 = jnp.zeros((B, D), _F32)
    hs1 = jnp.zeros((B, D), _F32)
    cs0 = jnp.zeros((B, D), _F32)
    cs1 = jnp.zeros((B, D), _F32)
    q_star = jnp.zeros((B, 2 * D), _F32)

    for _ in range(S2S_ITERS):
        ig = dot(q_star, w_ref['wi0_i'][...]) + dot(hs0, w_ref['wh0_i'][...]) + w_ref['b0_i'][...]
        fg = dot(q_star, w_ref['wi0_f'][...]) + dot(hs0, w_ref['wh0_f'][...]) + w_ref['b0_f'][...]
        gg = dot(q_star, w_ref['wi0_g'][...]) + dot(hs0, w_ref['wh0_g'][...]) + w_ref['b0_g'][...]
        og = dot(q_star, w_ref['wi0_o'][...]) + dot(hs0, w_ref['wh0_o'][...]) + w_ref['b0_o'][...]
        cs0 = jax.nn.sigmoid(fg) * cs0 + jax.nn.sigmoid(ig) * jnp.tanh(gg)
        hs0 = jax.nn.sigmoid(og) * jnp.tanh(cs0)

        ig = dot(hs0, w_ref['wi1_i'][...]) + dot(hs1, w_ref['wh1_i'][...]) + w_ref['b1_i'][...]
        fg = dot(hs0, w_ref['wi1_f'][...]) + dot(hs1, w_ref['wh1_f'][...]) + w_ref['b1_f'][...]
        gg = dot(hs0, w_ref['wi1_g'][...]) + dot(hs1, w_ref['wh1_g'][...]) + w_ref['b1_g'][...]
        og = dot(hs0, w_ref['wi1_o'][...]) + dot(hs1, w_ref['wh1_o'][...]) + w_ref['b1_o'][...]
        cs1 = jax.nn.sigmoid(fg) * cs1 + jax.nn.sigmoid(ig) * jnp.tanh(gg)
        hs1 = jax.nn.sigmoid(og) * jnp.tanh(cs1)
        q = hs1

        q_g = dot(mask, q)
        e = jnp.sum(h * q_g, axis=1, keepdims=True)
        masked = jnp.where(mask > 0.0, e, neg)
        smax = jnp.max(masked, axis=0, keepdims=True)
        smax_g = jnp.sum(mask * smax, axis=1, keepdims=True)
        ex = jnp.exp(e - smax_g)
        den = dot(maskt, ex)
        den_g = dot(mask, den)
        alpha = ex / den_g
        readout = dot(maskt, alpha * h)
        q_star = jnp.concatenate([q, readout], axis=1)

    gfe = jnp.maximum(dot(gf_ref[...], w_ref['wg1'][...]) + w_ref['bg1'][...], 0.0)
    gfe = jnp.maximum(dot(gfe, w_ref['wg2'][...]) + w_ref['bg2'][...], 0.0)
    pre = jnp.maximum(dot(q_star, w_ref['wp1a'][...]) + dot(gfe, w_ref['wp1b'][...])
                      + w_ref['bp1'][...], 0.0)
    out_ref[...] = dot(pre, w_ref['wp2'][...]) + w_ref['bp2'][...]


def _s2s(h, ida, idb, g_feat, sw):
    return pl.pallas_call(
        _s2s_body,
        out_shape=jax.ShapeDtypeStruct((B, 1), _F32),
    )(h, ida, idb, g_feat, *[sw[k] for k in _SW_KEYS])


def kernel(node_feats, edge_feats, g_feat, params, edge_index, graph_ids):
    p = params
    src2d = edge_index[0].astype(jnp.int32).reshape(NW * NCH, ICH)
    dst2d = edge_index[1].astype(jnp.int32).reshape(NW * NCH, ICH)
    ida = graph_ids.astype(jnp.int32).reshape(N, 1)
    idb = graph_ids.astype(jnp.int32).reshape(1, N)

    wih_t = p['gru_Wih'].T
    whh_t = p['gru_Whh'].T
    gw = {
        'wir': wih_t[:, 0 * D:1 * D], 'wiz': wih_t[:, 1 * D:2 * D], 'win': wih_t[:, 2 * D:3 * D],
        'whr': whh_t[:, 0 * D:1 * D], 'whz': whh_t[:, 1 * D:2 * D], 'whn': whh_t[:, 2 * D:3 * D],
        'bc': p['b_conv'].reshape(1, D),
        'br': (p['gru_bih'][0 * D:1 * D] + p['gru_bhh'][0 * D:1 * D]).reshape(1, D),
        'bz': (p['gru_bih'][1 * D:2 * D] + p['gru_bhh'][1 * D:2 * D]).reshape(1, D),
        'bi_n': p['gru_bih'][2 * D:3 * D].reshape(1, D),
        'bh_n': p['gru_bhh'][2 * D:3 * D].reshape(1, D),
    }

    def lstm_split(wih, whh, bih, bhh, gate):
        g = {'i': 0, 'f': 1, 'g': 2, 'o': 3}[gate]
        return (wih[g * D:(g + 1) * D].T, whh[g * D:(g + 1) * D].T,
                (bih[g * D:(g + 1) * D] + bhh[g * D:(g + 1) * D]).reshape(1, D))

    sw = {}
    for gate in 'ifgo':
        wi, wh, b = lstm_split(p['lstm_Wih0'], p['lstm_Whh0'], p['lstm_bih0'], p['lstm_bhh0'], gate)
        sw['wi0_' + gate], sw['wh0_' + gate], sw['b0_' + gate] = wi, wh, b
        wi, wh, b = lstm_split(p['lstm_Wih1'], p['lstm_Whh1'], p['lstm_bih1'], p['lstm_bhh1'], gate)
        sw['wi1_' + gate], sw['wh1_' + gate], sw['b1_' + gate] = wi, wh, b
    sw['wg1'] = p['W_g1']
    sw['bg1'] = p['b_g1'].reshape(1, DGH)
    sw['wg2'] = p['W_g2']
    sw['bg2'] = p['b_g2'].reshape(1, DGH)
    sw['wp1a'] = p['W_p1'][:2 * D]
    sw['wp1b'] = p['W_p1'][2 * D:]
    sw['bp1'] = p['b_p1'].reshape(1, D)
    sw['wp2'] = p['W_p2']
    sw['bp2'] = p['b_p2'].reshape(1, 1)

    hf, hb = _h0(node_feats, p['W_proj'], p['b_proj'].reshape(1, D))

    rmat = _RMAT
    zeros_nd = _ZEROS
    theta = None
    for s in range(STEPS):
        h_src = _gather(hb, src2d)
        if s == 0:
            theta, msg = _msg1(edge_feats, p['W_e1'], p['b_e1'].reshape(1, DEH),
                               p['W_e2'], p['b_e2'].reshape(1, D * D), h_src, rmat)
        else:
            msg = _msg(h_src, theta, rmat)
        agg2 = _scatter(msg, dst2d, zeros_nd)
        hf, hb = _gru(agg2, hf, gw)

    return _s2s(hf, ida, idb, g_feat, sw)


# speedup vs baseline: 1.1172x; 1.0112x over previous
import jax
import jax.numpy as jnp
import numpy as np
from jax import lax
from jax.experimental import pallas as pl
from jax.experimental.pallas import tpu as pltpu
from jax.experimental.pallas import tpu_sc as plsc

N, E, B = 4096, 16384, 128
DIN, DE, DG, D, DEH, DGH = 128, 16, 256, 64, 128, 512
STEPS, S2S_ITERS = 3, 3

NC, NS = 2, 16
NW = NC * NS
EPW = E // NW
ICH = 128
NCH = EPW // ICH
RPZ = N // NS

_F32 = jnp.float32
_BF16 = jnp.bfloat16

_sc_mesh = plsc.VectorSubcoreMesh(core_axis_name="c", subcore_axis_name="s")

_RMAT = np.repeat(np.eye(D, dtype=np.float32), D, axis=1)
_ZEROS = np.zeros((N, 2 * D), np.float32)


def _sc_gather_body(h_hbm, src_hbm, out_hbm, idx_v, rows_v, sem):
    wid = lax.axis_index("s") * NC + lax.axis_index("c")
    pltpu.sync_copy(src_hbm.at[pl.ds(wid * NCH, NCH)], idx_v)
    copies = [
        pltpu.async_copy(h_hbm.at[idx_v.at[j]], rows_v.at[pl.ds(j * ICH, ICH)], sem)
        for j in range(NCH)
    ]
    for c in copies:
        c.wait()
    pltpu.sync_copy(rows_v, out_hbm.at[pl.ds(wid * EPW, EPW)])


def _gather(h, src2d):
    fn = pl.kernel(
        _sc_gather_body,
        out_type=jax.ShapeDtypeStruct((E, 2 * D), _F32),
        mesh=_sc_mesh,
        scratch_types=[
            pltpu.VMEM((NCH, ICH), jnp.int32),
            pltpu.VMEM((EPW, 2 * D), _F32),
            pltpu.SemaphoreType.DMA,
        ],
    )
    return fn(h, src2d)


def _sc_scatter_body(msg_hbm, dst_hbm, zero_hbm, out_hbm, idx_v, rows_v, acc_sh, sem):
    cid = lax.axis_index("c")
    sid = lax.axis_index("s")
    wid = sid * NC + cid
    loads = [
        pltpu.async_copy(zero_hbm.at[pl.ds(sid * RPZ, RPZ)],
                         acc_sh.at[pl.ds(sid * RPZ, RPZ)], sem),
        pltpu.async_copy(dst_hbm.at[pl.ds(wid * NCH, NCH)], idx_v, sem),
        pltpu.async_copy(msg_hbm.at[pl.ds(wid * EPW, EPW)], rows_v, sem),
    ]
    for c in loads:
        c.wait()
    plsc.subcore_barrier()
    adds = [
        pltpu.async_copy(rows_v.at[pl.ds(j * ICH, ICH)], acc_sh.at[idx_v.at[j]],
                         sem, add=True)
        for j in range(NCH)
    ]
    for c in adds:
        c.wait()
    plsc.subcore_barrier()
    pltpu.sync_copy(acc_sh.at[pl.ds(sid * RPZ, RPZ)],
                    out_hbm.at[pl.ds(cid * N + sid * RPZ, RPZ)])


def _scatter(msg, dst2d, zeros_nd):
    fn = pl.kernel(
        _sc_scatter_body,
        out_type=jax.ShapeDtypeStruct((2 * N, 2 * D), _F32),
        mesh=_sc_mesh,
        scratch_types=[
            pltpu.VMEM((NCH, ICH), jnp.int32),
            pltpu.VMEM((EPW, 2 * D), _F32),
            pltpu.VMEM_SHARED((N, 2 * D), _F32),
            pltpu.SemaphoreType.DMA,
        ],
    )
    return fn(msg, dst2d, zeros_nd)


def _h0_body(nf_ref, wp_ref, bp_ref, hf_ref, hb_ref):
    h0 = jnp.maximum(
        jnp.dot(nf_ref[...], wp_ref[...], preferred_element_type=_F32)
        + bp_ref[...], 0.0)
    hf_ref[...] = h0
    hb_ref[...] = jnp.concatenate([h0, jnp.zeros_like(h0)], axis=1)


def _h0(node_feats, wp, bp):
    return pl.pallas_call(
        _h0_body,
        out_shape=(jax.ShapeDtypeStruct((N, D), _F32),
                   jax.ShapeDtypeStruct((N, 2 * D), _F32)),
    )(node_feats, wp, bp)


ET = 2048
ET1 = 1024


def _msg1_body(ef_ref, we1_ref, be1_ref, w2_ref, b2_ref, hs_ref, rmat_ref,
               th_ref, out_ref):
    t = jnp.maximum(
        jnp.dot(ef_ref[...], we1_ref[...], preferred_element_type=_F32)
        + be1_ref[...], 0.0)
    th = jnp.dot(t, w2_ref[...], preferred_element_type=_F32) + b2_ref[...]
    th_ref[...] = th.astype(_BF16)
    hs = hs_ref[...][:, :D]
    hexp = jnp.dot(hs, rmat_ref[...], preferred_element_type=_F32)
    parts = [hexp[:, j * 2 * D:(j + 1) * 2 * D] * th[:, j * 2 * D:(j + 1) * 2 * D]
             for j in range(D // 2)]
    while len(parts) > 1:
        parts = [parts[i] + parts[i + 1] for i in range(0, len(parts), 2)]
    red = parts[0]
    msg = red[:, :D] + red[:, D:]
    out_ref[...] = jnp.concatenate([msg, jnp.zeros_like(msg)], axis=1)


def _msg1(edge_feats, we1, be1, w2, b2row, h_src, rmat):
    return pl.pallas_call(
        _msg1_body,
        grid=(E // ET1,),
        in_specs=[pl.BlockSpec((ET1, DE), lambda i: (i, 0)),
                  pl.BlockSpec((DE, DEH), lambda i: (0, 0)),
                  pl.BlockSpec((1, DEH), lambda i: (0, 0)),
                  pl.BlockSpec((DEH, D * D), lambda i: (0, 0)),
                  pl.BlockSpec((1, D * D), lambda i: (0, 0)),
                  pl.BlockSpec((ET1, 2 * D), lambda i: (i, 0)),
                  pl.BlockSpec((D, D * D), lambda i: (0, 0))],
        out_specs=(pl.BlockSpec((ET1, D * D), lambda i: (i, 0)),
                   pl.BlockSpec((ET1, 2 * D), lambda i: (i, 0))),
        out_shape=(jax.ShapeDtypeStruct((E, D * D), _BF16),
                   jax.ShapeDtypeStruct((E, 2 * D), _F32)),
    )(edge_feats, we1, be1, w2, b2row, h_src, rmat)


def _msg_body(hs_ref, th_ref, rmat_ref, out_ref):
    hs = hs_ref[...][:, :D]
    hexp = jnp.dot(hs, rmat_ref[...], preferred_element_type=_F32)
    parts = [hexp[:, j * 2 * D:(j + 1) * 2 * D]
             * th_ref[:, j * 2 * D:(j + 1) * 2 * D].astype(_F32)
             for j in range(D // 2)]
    while len(parts) > 1:
        parts = [parts[i] + parts[i + 1] for i in range(0, len(parts), 2)]
    red = parts[0]
    msg = red[:, :D] + red[:, D:]
    out_ref[...] = jnp.concatenate([msg, jnp.zeros_like(msg)], axis=1)


def _msg(h_src, theta, rmat):
    return pl.pallas_call(
        _msg_body,
        grid=(E // ET,),
        in_specs=[pl.BlockSpec((ET, 2 * D), lambda i: (i, 0)),
                  pl.BlockSpec((ET, D * D), lambda i: (i, 0)),
                  pl.BlockSpec((D, D * D), lambda i: (0, 0))],
        out_specs=pl.BlockSpec((ET, 2 * D), lambda i: (i, 0)),
        out_shape=jax.ShapeDtypeStruct((E, 2 * D), _F32),
    )(h_src, theta, rmat)


RT = 1024


def _gru_body(a0_ref, a1_ref, h_ref, wir, wiz, win, whr, whz, whn,
              bc, br, bz, bi_n, bh_n, hf_ref, hb_ref):
    m = jnp.maximum(a0_ref[...][:, :D] + a1_ref[...][:, :D] + bc[...], 0.0)
    hv = h_ref[...]
    r = jax.nn.sigmoid(jnp.dot(m, wir[...], preferred_element_type=_F32)
                       + jnp.dot(hv, whr[...], preferred_element_type=_F32) + br[...])
    z = jax.nn.sigmoid(jnp.dot(m, wiz[...], preferred_element_type=_F32)
                       + jnp.dot(hv, whz[...], preferred_element_type=_F32) + bz[...])
    hn = jnp.dot(hv, whn[...], preferred_element_type=_F32) + bh_n[...]
    n = jnp.tanh(jnp.dot(m, win[...], preferred_element_type=_F32) + bi_n[...] + r * hn)
    hnew = (1.0 - z) * n + z * hv
    hf_ref[...] = hnew
    hb_ref[...] = jnp.concatenate([hnew, jnp.zeros_like(hnew)], axis=1)


def _gru(agg2, h, gw):
    full = pl.BlockSpec((D, D), lambda i: (0, 0))
    bias = pl.BlockSpec((1, D), lambda i: (0, 0))
    row = pl.BlockSpec((RT, D), lambda i: (i, 0))
    row2 = pl.BlockSpec((RT, 2 * D), lambda i: (i, 0))
    nsteps = N // RT
    row2hi = pl.BlockSpec((RT, 2 * D), lambda i: (i + nsteps, 0))
    return pl.pallas_call(
        _gru_body,
        grid=(nsteps,),
        in_specs=[row2, row2hi, row, full, full, full, full, full, full,
                  bias, bias, bias, bias, bias],
        out_specs=(row, row2),
        out_shape=(jax.ShapeDtypeStruct((N, D), _F32),
                   jax.ShapeDtypeStruct((N, 2 * D), _F32)),
    )(agg2, agg2, h, gw['wir'], gw['wiz'], gw['win'], gw['whr'], gw['whz'],
      gw['whn'], gw['bc'], gw['br'], gw['bz'], gw['bi_n'], gw['bh_n'])


_SW_KEYS = (
    ['wi0_' + g for g in 'ifgo'] + ['wh0_' + g for g in 'ifgo'] + ['b0_' + g for g in 'ifgo']
    + ['wi1_' + g for g in 'ifgo'] + ['wh1_' + g for g in 'ifgo'] + ['b1_' + g for g in 'ifgo']
    + ['wg1', 'bg1', 'wg2', 'bg2', 'wp1a', 'wp1b', 'bp1', 'wp2', 'bp2'])


def _s2s_body(h_ref, ida_ref, idb_ref, gf_ref, *rest):
    w_ref = dict(zip(_SW_KEYS, rest[:-1]))
    out_ref = rest[-1]
    h = h_ref[...]
    ida = ida_ref[...]
    idb = idb_ref[...]
    lane_b = lax.broadcasted_iota(jnp.int32, (N, B), 1)
    mask = (lane_b == ida).astype(_F32)
    sub_b = lax.broadcasted_iota(jnp.int32, (B, N), 0)
    maskt = (sub_b == idb).astype(_F32)
    neg = jnp.float32(-1e30)

    def dot(a, b):
        return jnp.dot(a, b, preferred_element_type=_F32)

    hs0 = jnp.zeros((B, D), _F32)
    hs1 = jnp.zeros((B, D), _F32)
    cs0 = jnp.zeros((B, D), _F32)
    cs1 = jnp.zeros((B, D), _F32)
    q_star = jnp.zeros((B, 2 * D), _F32)

    for _ in range(S2S_ITERS):
        ig = dot(q_star, w_ref['wi0_i'][...]) + dot(hs0, w_ref['wh0_i'][...]) + w_ref['b0_i'][...]
        fg = dot(q_star, w_ref['wi0_f'][...]) + dot(hs0, w_ref['wh0_f'][...]) + w_ref['b0_f'][...]
        gg = dot(q_star, w_ref['wi0_g'][...]) + dot(hs0, w_ref['wh0_g'][...]) + w_ref['b0_g'][...]
        og = dot(q_star, w_ref['wi0_o'][...]) + dot(hs0, w_ref['wh0_o'][...]) + w_ref['b0_o'][...]
        cs0 = jax.nn.sigmoid(fg) * cs0 + jax.nn.sigmoid(ig) * jnp.tanh(gg)
        hs0 = jax.nn.sigmoid(og) * jnp.tanh(cs0)

        ig = dot(hs0, w_ref['wi1_i'][...]) + dot(hs1, w_ref['wh1_i'][...]) + w_ref['b1_i'][...]
        fg = dot(hs0, w_ref['wi1_f'][...]) + dot(hs1, w_ref['wh1_f'][...]) + w_ref['b1_f'][...]
        gg = dot(hs0, w_ref['wi1_g'][...]) + dot(hs1, w_ref['wh1_g'][...]) + w_ref['b1_g'][...]
        og = dot(hs0, w_ref['wi1_o'][...]) + dot(hs1, w_ref['wh1_o'][...]) + w_ref['b1_o'][...]
        cs1 = jax.nn.sigmoid(fg) * cs1 + jax.nn.sigmoid(ig) * jnp.tanh(gg)
        hs1 = jax.nn.sigmoid(og) * jnp.tanh(cs1)
        q = hs1

        q_g = dot(mask, q)
        e = jnp.sum(h * q_g, axis=1, keepdims=True)
        masked = jnp.where(mask > 0.0, e, neg)
        smax = jnp.max(masked, axis=0, keepdims=True)
        smax_g = jnp.sum(mask * smax, axis=1, keepdims=True)
        ex = jnp.exp(e - smax_g)
        den = dot(maskt, ex)
        den_g = dot(mask, den)
        alpha = ex / den_g
        readout = dot(maskt, alpha * h)
        q_star = jnp.concatenate([q, readout], axis=1)

    gfe = jnp.maximum(dot(gf_ref[...], w_ref['wg1'][...]) + w_ref['bg1'][...], 0.0)
    gfe = jnp.maximum(dot(gfe, w_ref['wg2'][...]) + w_ref['bg2'][...], 0.0)
    pre = jnp.maximum(dot(q_star, w_ref['wp1a'][...]) + dot(gfe, w_ref['wp1b'][...])
                      + w_ref['bp1'][...], 0.0)
    out_ref[...] = dot(pre, w_ref['wp2'][...]) + w_ref['bp2'][...]


def _s2s(h, ida, idb, g_feat, sw):
    return pl.pallas_call(
        _s2s_body,
        out_shape=jax.ShapeDtypeStruct((B, 1), _F32),
    )(h, ida, idb, g_feat, *[sw[k] for k in _SW_KEYS])


def kernel(node_feats, edge_feats, g_feat, params, edge_index, graph_ids):
    p = params
    src2d = edge_index[0].astype(jnp.int32).reshape(NW * NCH, ICH)
    dst2d = edge_index[1].astype(jnp.int32).reshape(NW * NCH, ICH)
    ida = graph_ids.astype(jnp.int32).reshape(N, 1)
    idb = graph_ids.astype(jnp.int32).reshape(1, N)

    wih_t = p['gru_Wih'].T
    whh_t = p['gru_Whh'].T
    gw = {
        'wir': wih_t[:, 0 * D:1 * D], 'wiz': wih_t[:, 1 * D:2 * D], 'win': wih_t[:, 2 * D:3 * D],
        'whr': whh_t[:, 0 * D:1 * D], 'whz': whh_t[:, 1 * D:2 * D], 'whn': whh_t[:, 2 * D:3 * D],
        'bc': p['b_conv'].reshape(1, D),
        'br': (p['gru_bih'][0 * D:1 * D] + p['gru_bhh'][0 * D:1 * D]).reshape(1, D),
        'bz': (p['gru_bih'][1 * D:2 * D] + p['gru_bhh'][1 * D:2 * D]).reshape(1, D),
        'bi_n': p['gru_bih'][2 * D:3 * D].reshape(1, D),
        'bh_n': p['gru_bhh'][2 * D:3 * D].reshape(1, D),
    }

    def lstm_split(wih, whh, bih, bhh, gate):
        g = {'i': 0, 'f': 1, 'g': 2, 'o': 3}[gate]
        return (wih[g * D:(g + 1) * D].T, whh[g * D:(g + 1) * D].T,
                (bih[g * D:(g + 1) * D] + bhh[g * D:(g + 1) * D]).reshape(1, D))

    sw = {}
    for gate in 'ifgo':
        wi, wh, b = lstm_split(p['lstm_Wih0'], p['lstm_Whh0'], p['lstm_bih0'], p['lstm_bhh0'], gate)
        sw['wi0_' + gate], sw['wh0_' + gate], sw['b0_' + gate] = wi, wh, b
        wi, wh, b = lstm_split(p['lstm_Wih1'], p['lstm_Whh1'], p['lstm_bih1'], p['lstm_bhh1'], gate)
        sw['wi1_' + gate], sw['wh1_' + gate], sw['b1_' + gate] = wi, wh, b
    sw['wg1'] = p['W_g1']
    sw['bg1'] = p['b_g1'].reshape(1, DGH)
    sw['wg2'] = p['W_g2']
    sw['bg2'] = p['b_g2'].reshape(1, DGH)
    sw['wp1a'] = p['W_p1'][:2 * D]
    sw['wp1b'] = p['W_p1'][2 * D:]
    sw['bp1'] = p['b_p1'].reshape(1, D)
    sw['wp2'] = p['W_p2']
    sw['bp2'] = p['b_p2'].reshape(1, 1)

    hf, hb = _h0(node_feats, p['W_proj'], p['b_proj'].reshape(1, D))

    rmat = _RMAT
    zeros_nd = _ZEROS
    theta = None
    for s in range(STEPS):
        h_src = _gather(hb, src2d)
        if s == 0:
            theta, msg = _msg1(edge_feats, p['W_e1'], p['b_e1'].reshape(1, DEH),
                               p['W_e2'], p['b_e2'].reshape(1, D * D), h_src, rmat)
        else:
            msg = _msg(h_src, theta, rmat)
        agg2 = _scatter(msg, dst2d, zeros_nd)
        hf, hb = _gru(agg2, hf, gw)

    return _s2s(hf, ida, idb, g_feat, sw)
